# baseline (speedup 1.0000x reference)
import functools

import jax
import jax.numpy as jnp
from jax import lax
from jax.experimental import pallas as pl
from jax.experimental.pallas import tpu as pltpu
from jax.experimental.pallas import tpu_sc as plsc

_VOCAB = 1000000
_D = 64
_SEQ = 200
_BATCH = 1024
_L = 16

_NC = 2
_NS = 16
_NW = _NC * _NS
_SEQ_PER_W = _BATCH // _NW
_ROWS_PER_W = _SEQ_PER_W * _SEQ
_CSEQ = 2
_CROWS = _CSEQ * _SEQ
_NCHUNK = _SEQ_PER_W // _CSEQ


def _build_sc_call():
    mesh = plsc.VectorSubcoreMesh(core_axis_name="c", subcore_axis_name="s")

    @functools.partial(
        pl.kernel,
        mesh=mesh,
        compiler_params=pltpu.CompilerParams(use_tc_tiling_on_sc=False),
        out_type=jax.ShapeDtypeStruct((_BATCH * _SEQ, _D), jnp.float32),
        scratch_types=[
            pltpu.VMEM((_ROWS_PER_W,), jnp.int32),
            pltpu.VMEM((_ROWS_PER_W,), jnp.int32),
            pltpu.VMEM((2, _CROWS, _D), jnp.float32),
            pltpu.SemaphoreType.DMA,
            pltpu.SemaphoreType.DMA,
            pltpu.SemaphoreType.DMA,
            pltpu.SemaphoreType.DMA,
            pltpu.SemaphoreType.DMA,
        ],
    )
    def sc_gather(ids_hbm, remap_hbm, emb_hbm, wpe_hbm, out_hbm,
                  ids_v, dict_v, rows_v,
                  sem_r, sem_e0, sem_e1, sem_o0, sem_o1):
        wid = lax.axis_index("s") * _NC + lax.axis_index("c")
        row0 = wid * _ROWS_PER_W
        sem_e = (sem_e0, sem_e1)
        sem_o = (sem_o0, sem_o1)

        pltpu.sync_copy(ids_hbm.at[pl.ds(row0, _ROWS_PER_W)], ids_v)
        pltpu.async_copy(remap_hbm.at[ids_v], dict_v, sem_r).wait()

        def dict_slice(i):
            return dict_v.at[pl.ds(i * _CROWS, _CROWS)]

        def prefill(b):
            pltpu.sync_copy(wpe_hbm, rows_v.at[b])

        def start_emb(i, b):
            pltpu.make_async_copy(
                emb_hbm.at[dict_slice(i)], rows_v.at[b], sem_e[b],
            ).start(add=True)

        def wait_emb(i, b):
            pltpu.make_async_copy(
                emb_hbm.at[dict_slice(i)], rows_v.at[b], sem_e[b],
            ).wait()

        def start_out(i, b):
            base = row0 + i * _CROWS
            pltpu.make_async_copy(
                rows_v.at[b], out_hbm.at[pl.ds(base, _CROWS)], sem_o[b]).start()

        def wait_out(i, b):
            base = row0 + i * _CROWS
            pltpu.make_async_copy(
                rows_v.at[b], out_hbm.at[pl.ds(base, _CROWS)], sem_o[b]).wait()

        prefill(0)
        start_emb(0, 0)

        def step(i, b):
            wait_emb(i, b)

            @pl.when(i + 1 < _NCHUNK)
            def _():
                @pl.when(i >= 1)
                def _():
                    wait_out(i - 1, 1 - b)
                prefill(1 - b)
                start_emb(i + 1, 1 - b)

            start_out(i, b)

        def pair(g, carry):
            step(2 * g, 0)
            step(2 * g + 1, 1)
            return carry

        lax.fori_loop(0, _NCHUNK // 2, pair, 0)

        wait_out(_NCHUNK - 2, 0)
        wait_out(_NCHUNK - 1, 1)

    return sc_gather


_SC_CALL = _build_sc_call()


def kernel(input_ids, attention_mask, embedding_dict, input_ids2dict_ids, wpe):
    ids_flat = input_ids.reshape(_BATCH * _SEQ)
    wpe2 = jnp.concatenate([wpe] * _CSEQ, axis=0)
    out_flat = _SC_CALL(ids_flat, input_ids2dict_ids, embedding_dict, wpe2)
    return out_flat.reshape(_BATCH, _SEQ, _D), attention_mask
